# baseline (speedup 1.0000x reference)
import jax
import jax.numpy as jnp
from jax.experimental import pallas as pl
from jax.experimental.pallas import tpu as pltpu

_BM = 400


def _gcn_kernel(seq_hbm, wt_ref, adj_hbm, bias_ref, a_ref, o_hbm,
                seqv, fts, abuf, obuf, seq_sem, adj_sems, out_sems):
    n = seq_hbm.shape[0]
    nblk = n // _BM

    def adj_copy(i):
        return pltpu.make_async_copy(
            adj_hbm.at[pl.ds(i * _BM, _BM), :],
            abuf.at[i % 2],
            adj_sems.at[i % 2],
        )

    def out_copy(i):
        return pltpu.make_async_copy(
            obuf.at[i % 2],
            o_hbm.at[pl.ds(i * _BM, _BM), :],
            out_sems.at[i % 2],
        )

    seq_cp = pltpu.make_async_copy(seq_hbm, seqv, seq_sem)
    seq_cp.start()
    adj_copy(0).start()
    adj_copy(1).start()
    seq_cp.wait()
    fts[...] = jnp.dot(
        seqv[...], wt_ref[...], preferred_element_type=jnp.float32
    )
    a = a_ref[0, 0]

    for i in range(nblk):
        s = i % 2
        adj_copy(i).wait()
        out = jnp.dot(
            abuf[s], fts[...], preferred_element_type=jnp.float32
        ) + bias_ref[...]
        if i >= 2:
            out_copy(i - 2).wait()
        obuf[s] = jnp.where(out > 0, out, a * out)
        out_copy(i).start()
        if i + 2 < nblk:
            adj_copy(i + 2).start()

    out_copy(nblk - 2).wait()
    out_copy(nblk - 1).wait()


def kernel(seq, adj, W, bias, prelu_a):
    n, d_in = seq.shape
    d_out = W.shape[0]

    out = pl.pallas_call(
        _gcn_kernel,
        in_specs=[
            pl.BlockSpec(memory_space=pl.ANY),
            pl.BlockSpec(memory_space=pltpu.VMEM),
            pl.BlockSpec(memory_space=pl.ANY),
            pl.BlockSpec(memory_space=pltpu.VMEM),
            pl.BlockSpec(memory_space=pltpu.VMEM),
        ],
        out_specs=pl.BlockSpec(memory_space=pl.ANY),
        out_shape=jax.ShapeDtypeStruct((n, d_out), jnp.float32),
        scratch_shapes=[
            pltpu.VMEM((n, d_in), jnp.float32),
            pltpu.VMEM((n, d_out), jnp.float32),
            pltpu.VMEM((2, _BM, n), jnp.float32),
            pltpu.VMEM((2, _BM, d_out), jnp.float32),
            pltpu.SemaphoreType.DMA,
            pltpu.SemaphoreType.DMA((2,)),
            pltpu.SemaphoreType.DMA((2,)),
        ],
        compiler_params=pltpu.CompilerParams(
            vmem_limit_bytes=62 * 1024 * 1024,
        ),
    )(seq, W.T, adj, bias.reshape(1, d_out), prelu_a.reshape(1, 1))
    return out


# speedup vs baseline: 1.0148x; 1.0148x over previous
import jax
import jax.numpy as jnp
from jax.experimental import pallas as pl
from jax.experimental.pallas import tpu as pltpu

_BM = 400


def _gcn_kernel(seq_hbm, wt_ref, adj_hbm, bias_ref, a_ref, o_hbm,
                seqv, fts, abuf0, abuf1, obuf0, obuf1,
                seq_sem, asem0, asem1, osem0, osem1):
    n = seq_hbm.shape[0]
    nblk = n // _BM
    abufs = (abuf0, abuf1)
    obufs = (obuf0, obuf1)
    asems = (asem0, asem1)
    osems = (osem0, osem1)

    def adj_copy(i):
        return pltpu.make_async_copy(
            adj_hbm.at[pl.ds(i * _BM, _BM), :], abufs[i % 2], asems[i % 2]
        )

    def out_copy(i):
        return pltpu.make_async_copy(
            obufs[i % 2], o_hbm.at[pl.ds(i * _BM, _BM), :], osems[i % 2]
        )

    seq_cp = pltpu.make_async_copy(seq_hbm, seqv, seq_sem)
    seq_cp.start()
    adj_copy(0).start()
    adj_copy(1).start()
    seq_cp.wait()
    fts[...] = jnp.dot(
        seqv[...], wt_ref[...], preferred_element_type=jnp.float32
    )
    a = a_ref[0, 0]

    for i in range(nblk):
        s = i % 2
        adj_copy(i).wait()
        out = jnp.dot(
            abufs[s][...], fts[...], preferred_element_type=jnp.float32
        ) + bias_ref[...]
        if i >= 2:
            out_copy(i - 2).wait()
        obufs[s][...] = jnp.where(out > 0, out, a * out)
        out_copy(i).start()
        if i + 2 < nblk:
            adj_copy(i + 2).start()

    out_copy(nblk - 2).wait()
    out_copy(nblk - 1).wait()


def kernel(seq, adj, W, bias, prelu_a):
    n, d_in = seq.shape
    d_out = W.shape[0]

    out = pl.pallas_call(
        _gcn_kernel,
        in_specs=[
            pl.BlockSpec(memory_space=pl.ANY),
            pl.BlockSpec(memory_space=pltpu.VMEM),
            pl.BlockSpec(memory_space=pl.ANY),
            pl.BlockSpec(memory_space=pltpu.VMEM),
            pl.BlockSpec(memory_space=pltpu.VMEM),
        ],
        out_specs=pl.BlockSpec(memory_space=pl.ANY),
        out_shape=jax.ShapeDtypeStruct((n, d_out), jnp.float32),
        scratch_shapes=[
            pltpu.VMEM((n, d_in), jnp.float32),
            pltpu.VMEM((n, d_out), jnp.float32),
            pltpu.VMEM((_BM, n), jnp.float32),
            pltpu.VMEM((_BM, n), jnp.float32),
            pltpu.VMEM((_BM, d_out), jnp.float32),
            pltpu.VMEM((_BM, d_out), jnp.float32),
            pltpu.SemaphoreType.DMA,
            pltpu.SemaphoreType.DMA,
            pltpu.SemaphoreType.DMA,
            pltpu.SemaphoreType.DMA,
            pltpu.SemaphoreType.DMA,
        ],
        compiler_params=pltpu.CompilerParams(
            vmem_limit_bytes=62 * 1024 * 1024,
        ),
    )(seq, W.T, adj, bias.reshape(1, d_out), prelu_a.reshape(1, 1))
    return out


# speedup vs baseline: 1.1018x; 1.0857x over previous
import jax
import jax.numpy as jnp
from jax.experimental import pallas as pl
from jax.experimental.pallas import tpu as pltpu

_BM = 400


def _gcn_kernel(seq_ref, wt_ref, adj_ref, bias_ref, a_ref, o_ref, fts_ref):
    @pl.when(pl.program_id(0) == 0)
    def _():
        fts_ref[...] = jnp.dot(
            seq_ref[...], wt_ref[...], preferred_element_type=jnp.float32
        )

    out = jnp.dot(
        adj_ref[...], fts_ref[...], preferred_element_type=jnp.float32
    ) + bias_ref[...]
    o_ref[...] = jnp.where(out > 0, out, a_ref[0, 0] * out)


def kernel(seq, adj, W, bias, prelu_a):
    n, d_in = seq.shape
    d_out = W.shape[0]

    out = pl.pallas_call(
        _gcn_kernel,
        grid=(n // _BM,),
        in_specs=[
            pl.BlockSpec((n, d_in), lambda i: (0, 0)),
            pl.BlockSpec((d_in, d_out), lambda i: (0, 0)),
            pl.BlockSpec((_BM, n), lambda i: (i, 0)),
            pl.BlockSpec((1, d_out), lambda i: (0, 0)),
            pl.BlockSpec((1, 1), lambda i: (0, 0)),
        ],
        out_specs=pl.BlockSpec((_BM, d_out), lambda i: (i, 0)),
        out_shape=jax.ShapeDtypeStruct((n, d_out), jnp.float32),
        scratch_shapes=[pltpu.VMEM((n, d_out), jnp.float32)],
        compiler_params=pltpu.CompilerParams(
            dimension_semantics=("arbitrary",),
            vmem_limit_bytes=62 * 1024 * 1024,
        ),
    )(seq, W.T, adj, bias.reshape(1, d_out), prelu_a.reshape(1, 1))
    return out
